# baseline (speedup 1.0000x reference)
import functools

import jax
import jax.numpy as jnp
from jax import lax
from jax.experimental import pallas as pl
from jax.experimental.pallas import tpu as pltpu
from jax.experimental.pallas import tpu_sc as plsc

_B = 16384
_D = 32
_NC = 2
_NS = 16
_NW = _NC * _NS
_RPT = _B // _NW
_L = 16


@functools.lru_cache(maxsize=1)
def _sc_diff_kernel():
    mesh = plsc.VectorSubcoreMesh(core_axis_name="c", subcore_axis_name="s")

    @functools.partial(
        pl.kernel,
        mesh=mesh,
        compiler_params=pltpu.CompilerParams(
            needs_layout_passes=False, use_tc_tiling_on_sc=False
        ),
        out_type=jax.ShapeDtypeStruct((_B,), jnp.float32),
        scratch_types=[
            pltpu.VMEM((_RPT,), jnp.int32),
            pltpu.VMEM((_RPT,), jnp.int32),
            pltpu.VMEM((_RPT,), jnp.int32),
            pltpu.VMEM((_RPT, _D), jnp.float32),
            pltpu.VMEM((_RPT, _D), jnp.float32),
            pltpu.VMEM((_RPT, _D), jnp.float32),
            pltpu.VMEM((_RPT,), jnp.float32),
            pltpu.SemaphoreType.DMA,
        ],
    )
    def diff_kernel(users_hbm, items_hbm, negs_hbm, uw_hbm, iw_hbm, out_hbm,
                    u_idx, p_idx, n_idx, u_rows, p_rows, n_rows, out_v, sem):
        wid = lax.axis_index("s") * _NC + lax.axis_index("c")
        base = wid * _RPT
        pltpu.sync_copy(users_hbm.at[pl.ds(base, _RPT)], u_idx)
        pltpu.sync_copy(items_hbm.at[pl.ds(base, _RPT)], p_idx)
        pltpu.sync_copy(negs_hbm.at[pl.ds(base, _RPT)], n_idx)
        cu = pltpu.async_copy(uw_hbm.at[u_idx], u_rows, sem)
        cp = pltpu.async_copy(iw_hbm.at[p_idx], p_rows, sem)
        cn = pltpu.async_copy(iw_hbm.at[n_idx], n_rows, sem)
        cu.wait()
        cp.wait()
        cn.wait()

        def blk_body(blk, carry):
            rows = blk * _L + lax.iota(jnp.int32, _L)
            acc = jnp.zeros((_L,), jnp.float32)
            for d in range(_D):
                dd = jnp.full((_L,), d, jnp.int32)
                u = plsc.load_gather(u_rows, [rows, dd])
                p = plsc.load_gather(p_rows, [rows, dd])
                n = plsc.load_gather(n_rows, [rows, dd])
                acc = acc + u * (p - n)
            plsc.store_scatter(out_v, [rows], acc)
            return carry

        lax.fori_loop(0, _RPT // _L, blk_body, 0)
        pltpu.sync_copy(out_v, out_hbm.at[pl.ds(base, _RPT)])

    return diff_kernel


def _tc_loss(diff2d):
    def body(x_ref, o_ref):
        x = x_ref[...]
        ls = jnp.minimum(x, 0.0) - jnp.log1p(jnp.exp(-jnp.abs(x)))
        o_ref[0, 0] = -(jnp.sum(ls) / _B)

    return pl.pallas_call(
        body,
        out_shape=jax.ShapeDtypeStruct((1, 1), jnp.float32),
        out_specs=pl.BlockSpec(memory_space=pltpu.SMEM),
    )(diff2d)


def kernel(batch, neg_items, users_weight, items_weight):
    users = batch[:, 0].astype(jnp.int32)
    items = batch[:, 2].astype(jnp.int32)
    negs = neg_items.astype(jnp.int32)
    diff = _sc_diff_kernel()(users, items, negs, users_weight, items_weight)
    loss = _tc_loss(diff.reshape(128, 128))
    return loss[0, 0]
